# baseline (speedup 1.0000x reference)
import functools

import jax
import jax.numpy as jnp
from jax import lax
from jax.experimental import pallas as pl
from jax.experimental.pallas import tpu as pltpu
from jax.experimental.pallas import tpu_sc as plsc

INPUT_DIM = 128
CONTEXT_DIM = 8
BATCH = 16384
NUM_CTX = 2 ** CONTEXT_DIM

NUM_CORES = 2
NUM_SUBCORES = 16
LANES = 16
NUM_WORKERS = NUM_CORES * NUM_SUBCORES
CHUNK = BATCH // NUM_WORKERS
HALF = CHUNK // 2

TC_BLOCK = 4096
D_BLOCK = 32


def _ctx_body(vt_ref, si_ref, b_ref, bc_ref, ctx_ref):
    proj = lax.dot_general(
        vt_ref[...], si_ref[...],
        dimension_numbers=(((1,), (0,)), ((), ())),
        preferred_element_type=jnp.float32,
    )
    bits = (proj > b_ref[...]).astype(jnp.float32)
    ctx_ref[...] = jnp.sum(bits * bc_ref[...], axis=0).astype(jnp.int32)


def _context_ids(side_information, vt, b, boolean_converter):
    side_dim = side_information.shape[0]
    return pl.pallas_call(
        _ctx_body,
        grid=(BATCH // TC_BLOCK,),
        in_specs=[
            pl.BlockSpec((CONTEXT_DIM, side_dim), lambda i: (0, 0)),
            pl.BlockSpec((side_dim, TC_BLOCK), lambda i: (0, i)),
            pl.BlockSpec((CONTEXT_DIM, 1), lambda i: (0, 0)),
            pl.BlockSpec((CONTEXT_DIM, 1), lambda i: (0, 0)),
        ],
        out_specs=pl.BlockSpec((TC_BLOCK,), lambda i: (i,)),
        out_shape=jax.ShapeDtypeStruct((BATCH,), jnp.int32),
    )(vt, side_information, b, boolean_converter)


def _gln_sc_body(w_hbm, ctx_hbm, lp_hbm, out_hbm,
                 w_ts, ctx_ts, lp_ts, out_ts, sem_w, sem_c, sem_l0, sem_l1):
    wid = lax.axis_index("s") * NUM_CORES + lax.axis_index("c")
    base = wid * CHUNK
    cp_w = pltpu.async_copy(w_hbm, w_ts, sem_w)
    cp_c = pltpu.async_copy(ctx_hbm.at[pl.ds(base, CHUNK)], ctx_ts, sem_c)
    cp_l0 = pltpu.async_copy(
        lp_hbm.at[:, pl.ds(base, HALF)], lp_ts.at[:, 0:HALF], sem_l0)
    cp_l1 = pltpu.async_copy(
        lp_hbm.at[:, pl.ds(base + HALF, HALF)], lp_ts.at[:, HALF:CHUNK], sem_l1)

    def group(j, carry):
        col = j * LANES
        cvec = ctx_ts[pl.ds(col, LANES)]

        def dblock(k, accs):
            a0, a1, a2, a3 = accs
            acc4 = [a0, a1, a2, a3]
            for dd in range(D_BLOCK):
                d = k * D_BLOCK + dd
                w = plsc.load_gather(w_ts, [jnp.full((LANES,), 0, jnp.int32) + d,
                                            cvec])
                x = lp_ts[d, pl.ds(col, LANES)]
                acc4[dd % 4] = acc4[dd % 4] + w * x
            return tuple(acc4)

        zero = jnp.zeros((LANES,), jnp.float32)
        a0, a1, a2, a3 = lax.fori_loop(
            0, INPUT_DIM // D_BLOCK, dblock, (zero, zero, zero, zero))
        out_ts[pl.ds(col, LANES)] = (a0 + a1) + (a2 + a3)
        return carry

    cp_w.wait()
    cp_c.wait()
    cp_l0.wait()
    lax.fori_loop(0, HALF // LANES, group, 0)
    cp_l1.wait()
    lax.fori_loop(HALF // LANES, CHUNK // LANES, group, 0)
    pltpu.sync_copy(out_ts, out_hbm.at[pl.ds(base, CHUNK)])


@functools.cache
def _gln_sc():
    return pl.kernel(
        _gln_sc_body,
        out_type=jax.ShapeDtypeStruct((BATCH,), jnp.float32),
        mesh=plsc.VectorSubcoreMesh(
            core_axis_name="c", subcore_axis_name="s",
            num_cores=NUM_CORES, num_subcores=NUM_SUBCORES,
        ),
        scratch_types=[
            pltpu.VMEM((INPUT_DIM, NUM_CTX), jnp.float32),
            pltpu.VMEM((CHUNK,), jnp.int32),
            pltpu.VMEM((INPUT_DIM, CHUNK), jnp.float32),
            pltpu.VMEM((CHUNK,), jnp.float32),
            pltpu.SemaphoreType.DMA,
            pltpu.SemaphoreType.DMA,
            pltpu.SemaphoreType.DMA,
            pltpu.SemaphoreType.DMA,
        ],
        compiler_params=pltpu.CompilerParams(
            use_tc_tiling_on_sc=False, needs_layout_passes=False,
        ),
    )


def kernel(logit_previous, side_information, v, b, weights, boolean_converter):
    ctx = _context_ids(side_information, v.T, b, boolean_converter)
    return _gln_sc()(weights, ctx, logit_previous)


# speedup vs baseline: 1.0782x; 1.0782x over previous
import functools

import jax
import jax.numpy as jnp
from jax import lax
from jax.experimental import pallas as pl
from jax.experimental.pallas import tpu as pltpu
from jax.experimental.pallas import tpu_sc as plsc

INPUT_DIM = 128
CONTEXT_DIM = 8
BATCH = 16384
NUM_CTX = 2 ** CONTEXT_DIM

NUM_CORES = 2
NUM_SUBCORES = 16
LANES = 16
NUM_WORKERS = NUM_CORES * NUM_SUBCORES
CHUNK = BATCH // NUM_WORKERS
HALF = CHUNK // 2

TC_BLOCK = 4096
D_BLOCK = 32


def _ctx_body(vt_ref, si_ref, b_ref, bc_ref, ctx_ref):
    proj = lax.dot_general(
        vt_ref[...], si_ref[...],
        dimension_numbers=(((1,), (0,)), ((), ())),
        preferred_element_type=jnp.float32,
    )
    bits = (proj > b_ref[...]).astype(jnp.float32)
    ctx_ref[...] = jnp.sum(bits * bc_ref[...], axis=0).astype(jnp.int32)


def _context_ids(side_information, vt, b, boolean_converter):
    side_dim = side_information.shape[0]
    return pl.pallas_call(
        _ctx_body,
        grid=(BATCH // TC_BLOCK,),
        in_specs=[
            pl.BlockSpec((CONTEXT_DIM, side_dim), lambda i: (0, 0)),
            pl.BlockSpec((side_dim, TC_BLOCK), lambda i: (0, i)),
            pl.BlockSpec((CONTEXT_DIM, 1), lambda i: (0, 0)),
            pl.BlockSpec((CONTEXT_DIM, 1), lambda i: (0, 0)),
        ],
        out_specs=pl.BlockSpec((TC_BLOCK,), lambda i: (i,)),
        out_shape=jax.ShapeDtypeStruct((BATCH,), jnp.int32),
    )(vt, side_information, b, boolean_converter)


def _gln_sc_body(w_hbm, ctx_hbm, lp_hbm, out_hbm,
                 w_ts, ctx_ts, lp_ts, out_ts, sem_w, sem_c, sem_l0, sem_l1):
    wid = lax.axis_index("s") * NUM_CORES + lax.axis_index("c")
    base = wid * CHUNK
    cp_w = pltpu.async_copy(w_hbm, w_ts, sem_w)
    cp_c = pltpu.async_copy(ctx_hbm.at[pl.ds(base, CHUNK)], ctx_ts, sem_c)
    cp_l0 = pltpu.async_copy(
        lp_hbm.at[:, pl.ds(base, HALF)], lp_ts.at[:, 0:HALF], sem_l0)
    cp_l1 = pltpu.async_copy(
        lp_hbm.at[:, pl.ds(base + HALF, HALF)], lp_ts.at[:, HALF:CHUNK], sem_l1)

    def group(j, carry):
        col = j * LANES
        cvec = ctx_ts[pl.ds(col, LANES)]

        def dblock(k, accs):
            a0, a1, a2, a3 = accs
            acc4 = [a0, a1, a2, a3]
            for dd in range(D_BLOCK):
                d = k * D_BLOCK + dd
                w = plsc.load_gather(w_ts, [jnp.full((LANES,), 0, jnp.int32) + d,
                                            cvec])
                x = lp_ts[d, pl.ds(col, LANES)]
                acc4[dd % 4] = acc4[dd % 4] + w * x
            return tuple(acc4)

        zero = jnp.zeros((LANES,), jnp.float32)
        a0, a1, a2, a3 = lax.fori_loop(
            0, INPUT_DIM // D_BLOCK, dblock, (zero, zero, zero, zero))
        out_ts[pl.ds(col, LANES)] = (a0 + a1) + (a2 + a3)
        return carry

    cp_w.wait()
    cp_c.wait()
    cp_l0.wait()
    lax.fori_loop(0, HALF // LANES, group, 0)
    cp_l1.wait()
    lax.fori_loop(HALF // LANES, CHUNK // LANES, group, 0)
    pltpu.sync_copy(out_ts, out_hbm.at[pl.ds(base, CHUNK)])


@functools.cache
def _gln_sc():
    return pl.kernel(
        _gln_sc_body,
        out_type=jax.ShapeDtypeStruct((BATCH,), jnp.float32),
        mesh=plsc.VectorSubcoreMesh(
            core_axis_name="c", subcore_axis_name="s",
            num_cores=NUM_CORES, num_subcores=NUM_SUBCORES,
        ),
        scratch_types=[
            pltpu.VMEM((INPUT_DIM, NUM_CTX), jnp.float32),
            pltpu.VMEM((CHUNK,), jnp.int32),
            pltpu.VMEM((INPUT_DIM, CHUNK), jnp.float32),
            pltpu.VMEM((CHUNK,), jnp.float32),
            pltpu.SemaphoreType.DMA,
            pltpu.SemaphoreType.DMA,
            pltpu.SemaphoreType.DMA,
            pltpu.SemaphoreType.DMA,
        ],
        compiler_params=pltpu.CompilerParams(
            use_tc_tiling_on_sc=True, needs_layout_passes=False,
        ),
    )


def kernel(logit_previous, side_information, v, b, weights, boolean_converter):
    ctx = _context_ids(side_information, v.T, b, boolean_converter)
    return _gln_sc()(weights, ctx, logit_previous)
